# Initial kernel scaffold; baseline (speedup 1.0000x reference)
import jax
import jax.numpy as jnp
from jax.experimental import pallas as pl


def kernel(pc, idx, params, basis):
    raise NotImplementedError("write your pallas kernel here")


# speedup vs baseline: 1.1598x; 1.1598x over previous
import functools
from math import ceil

import jax
import jax.numpy as jnp
from jax import lax
from jax.experimental import pallas as pl

B, N, A = 8, 16384, 40
K = ceil(N / A)
KP = 416
HID = 192
HEADS = 6
DEPTH = 6
SH2 = 7
SH3 = 49
NT = B * A
CPB = 8
NEG = -1e30


def _bf(x):
    return x.astype(jnp.bfloat16).astype(jnp.float32)


def _pe_args(v, basis):
    return jnp.dot(_bf(v), _bf(basis), precision=lax.Precision.HIGHEST,
                   preferred_element_type=jnp.float32)


def _edge_kernel(nbr_ref, ctr_ref, basis_ref, w1_ref, b1_ref, w2_ref,
                 b2_ref, out_ref):
    basis = basis_ref[...]
    w1 = w1_ref[...]
    b1 = b1_ref[...]
    w2 = w2_ref[...]
    b2 = b2_ref[...]
    rowid = lax.broadcasted_iota(jnp.int32, (KP, 256), 0)
    valid = rowid < K
    for i in range(CPB):
        nbr = nbr_ref[0, i * KP:(i + 1) * KP, :]
        c = ctr_ref[0, i, :]
        rel = nbr - c
        p = _pe_args(rel, basis)
        feat = jnp.concatenate([rel, jnp.sin(p), jnp.cos(p)], axis=-1)
        l1 = jnp.dot(feat, w1, preferred_element_type=jnp.float32) + b1
        l1 = jnp.maximum(l1, 0.0)
        h = jnp.dot(l1, w2, preferred_element_type=jnp.float32) + b2
        h = jnp.where(valid, h, NEG)
        out_ref[0, i, :] = jnp.max(h, axis=0)


def _edge_agg(nbr, centers, basis, loc1_w, loc1_b, loc2_w, loc2_b):
    g = NT // CPB
    nbr_r = nbr.reshape(g, CPB * KP, 3)
    ctr_r = centers.reshape(g, CPB, 3)
    b1 = loc1_b.reshape(1, -1)
    b2 = loc2_b.reshape(1, -1)
    full = lambda a: pl.BlockSpec(a.shape, lambda i: (0,) * a.ndim)
    return pl.pallas_call(
        _edge_kernel,
        grid=(g,),
        in_specs=[
            pl.BlockSpec((1, CPB * KP, 3), lambda i: (i, 0, 0)),
            pl.BlockSpec((1, CPB, 3), lambda i: (i, 0, 0)),
            full(basis), full(loc1_w), full(b1), full(loc2_w), full(b2),
        ],
        out_specs=pl.BlockSpec((1, CPB, 256), lambda i: (i, 0, 0)),
        out_shape=jax.ShapeDtypeStruct((g, CPB, 256), jnp.float32),
    )(nbr_r, ctr_r, basis, loc1_w, b1, loc2_w, b2).reshape(NT, 256)


def _glob_kernel(agg_ref, w1_ref, b1_ref, w2_ref, b2_ref, out_ref):
    dot = functools.partial(jnp.dot, preferred_element_type=jnp.float32)
    x = jnp.maximum(dot(agg_ref[...], w1_ref[...]) + b1_ref[...], 0.0)
    out_ref[...] = dot(x, w2_ref[...]) + b2_ref[...]


def _glob(agg, w1, b1, w2, b2):
    return pl.pallas_call(
        _glob_kernel,
        out_shape=jax.ShapeDtypeStruct((NT, HID), jnp.float32),
    )(agg, w1, b1.reshape(1, -1), w2, b2.reshape(1, -1))


def _pe_embed(x, basis):
    p = x @ basis
    return jnp.concatenate([jnp.sin(p), jnp.cos(p)], axis=-1)


def _linear(x, w, b=None):
    y = x @ w
    return y + b if b is not None else y


def _layernorm(x, g, b, eps=1e-6):
    m = x.mean(-1, keepdims=True)
    v = ((x - m) ** 2).mean(-1, keepdims=True)
    return (x - m) / jnp.sqrt(v + eps) * g + b


def _attention(x, p):
    Bt, T, C = x.shape
    hd = C // HEADS
    qkv = _linear(x, p['qkv_w'], p['qkv_b']).reshape(
        Bt, T, 3, HEADS, hd).transpose(2, 0, 3, 1, 4)
    q, k, v = qkv[0], qkv[1], qkv[2]
    a = jax.nn.softmax((q @ k.transpose(0, 1, 3, 2)) * (hd ** -0.5), axis=-1)
    o = (a @ v).transpose(0, 2, 1, 3).reshape(Bt, T, C)
    return _linear(o, p['proj_w'], p['proj_b'])


def _block(x, p):
    h = x + p['g1'] * _attention(_layernorm(x, p['ln1_g'], p['ln1_b']), p)
    m = _linear(jax.nn.gelu(_linear(_layernorm(h, p['ln2_g'], p['ln2_b']),
                                    p['fc1_w'], p['fc1_b'])),
                p['fc2_w'], p['fc2_b'])
    return h + p['g2'] * m


def _transformer(x, pe, params):
    h = x + pe
    for p in params['blocks']:
        h = _block(h, p)
    return _layernorm(h, params['ln_f_g'], params['ln_f_b'])


def kernel(pc, idx, params, basis):
    pos = pc.reshape(B * N, 3)
    idx_flat = (idx + jnp.arange(B)[:, None] * N).reshape(-1)
    centers = pos[idx_flat]
    c = centers.reshape(B, A, 3)
    d2 = jnp.sum((c[:, :, None, :] - pc[:, None, :, :]) ** 2, axis=-1)
    _, nn_idx = lax.top_k(-d2, K)
    src = (nn_idx + jnp.arange(B)[:, None, None] * N).reshape(NT, K)
    nbr = pos[src]
    nbr = jnp.pad(nbr, ((0, 0), (0, KP - K), (0, 0)))

    agg = _edge_agg(nbr, centers, basis,
                    params['loc1_w'], params['loc1_b'],
                    params['loc2_w'], params['loc2_b'])
    x = _glob(agg, params['glob1_w'], params['glob1_b'],
              params['glob2_w'], params['glob2_b'])

    x = x.reshape(B, A, HID)
    center = centers.reshape(B, A, 3)

    def embed_in(v):
        return _linear(jnp.concatenate([v, _pe_embed(v, basis)], axis=-1),
                       params['embed_w'], params['embed_b'])

    ce = embed_in(center)
    x = _transformer(x, ce, params)
    dxyz = _linear(_layernorm(x, params['lnx_g'], params['lnx_b']),
                   params['xyz_w'])
    de = embed_in(dxyz)
    x = _transformer(x, ce + de, params)
    dtxyz = _linear(_layernorm(x, params['lnt_g'], params['lnt_b']),
                    params['txyz_w'])
    te = embed_in(dtxyz)
    x = _transformer(x, ce + de + te, params)
    sh2 = _linear(_layernorm(x, params['ln2d_g'], params['ln2d_b']),
                  params['sh2d_w'])
    s2e = _linear(sh2, params['sh2emb_w'], params['sh2emb_b'])
    x = _transformer(x, ce + de + te + s2e, params)
    sh3 = _linear(_layernorm(x, params['ln3d_g'], params['ln3d_b']),
                  params['sh3d_w'])
    return jnp.concatenate([center + dxyz, center + dtxyz, sh2, sh3],
                           axis=-1)


# speedup vs baseline: 1.1784x; 1.0160x over previous
import functools
from math import ceil

import jax
import jax.numpy as jnp
from jax import lax
from jax.experimental import pallas as pl

B, N, A = 8, 16384, 40
K = ceil(N / A)
KP = 416
HID = 192
HEADS = 6
DEPTH = 6
SH2 = 7
SH3 = 49
NT = B * A
CPB = 8
NEG = -1e30


def _bf(x):
    return x.astype(jnp.bfloat16).astype(jnp.float32)


def _pe_args(v, basis):
    return jnp.dot(_bf(v), _bf(basis), precision=lax.Precision.HIGHEST,
                   preferred_element_type=jnp.float32)


def _edge_kernel(nbr_ref, ctr_ref, basis_ref, w1_ref, b1_ref, w2_ref,
                 b2_ref, out_ref):
    basis = basis_ref[...]
    w1 = w1_ref[...]
    b1 = b1_ref[...]
    w2 = w2_ref[...]
    b2 = b2_ref[...]
    rowid = lax.broadcasted_iota(jnp.int32, (KP, 256), 0)
    valid = rowid < K
    rel = jnp.concatenate(
        [nbr_ref[0, i * KP:(i + 1) * KP, :] - ctr_ref[0, i, :]
         for i in range(CPB)], axis=0)
    p = _pe_args(rel, basis)
    feat = jnp.concatenate([rel, jnp.sin(p), jnp.cos(p)], axis=-1)
    l1 = jnp.dot(feat, w1, preferred_element_type=jnp.float32) + b1
    l1 = jnp.maximum(l1, 0.0)
    h = jnp.dot(l1, w2, preferred_element_type=jnp.float32) + b2
    for i in range(CPB):
        hc = jnp.where(valid, h[i * KP:(i + 1) * KP, :], NEG)
        out_ref[0, i, :] = jnp.max(hc, axis=0)


def _edge_agg(nbr, centers, basis, loc1_w, loc1_b, loc2_w, loc2_b):
    g = NT // CPB
    nbr_r = nbr.reshape(g, CPB * KP, 3)
    ctr_r = centers.reshape(g, CPB, 3)
    b1 = loc1_b.reshape(1, -1)
    b2 = loc2_b.reshape(1, -1)
    full = lambda a: pl.BlockSpec(a.shape, lambda i: (0,) * a.ndim)
    return pl.pallas_call(
        _edge_kernel,
        grid=(g,),
        in_specs=[
            pl.BlockSpec((1, CPB * KP, 3), lambda i: (i, 0, 0)),
            pl.BlockSpec((1, CPB, 3), lambda i: (i, 0, 0)),
            full(basis), full(loc1_w), full(b1), full(loc2_w), full(b2),
        ],
        out_specs=pl.BlockSpec((1, CPB, 256), lambda i: (i, 0, 0)),
        out_shape=jax.ShapeDtypeStruct((g, CPB, 256), jnp.float32),
    )(nbr_r, ctr_r, basis, loc1_w, b1, loc2_w, b2).reshape(NT, 256)


def _glob_kernel(agg_ref, w1_ref, b1_ref, w2_ref, b2_ref, out_ref):
    dot = functools.partial(jnp.dot, preferred_element_type=jnp.float32)
    x = jnp.maximum(dot(agg_ref[...], w1_ref[...]) + b1_ref[...], 0.0)
    out_ref[...] = dot(x, w2_ref[...]) + b2_ref[...]


def _glob(agg, w1, b1, w2, b2):
    return pl.pallas_call(
        _glob_kernel,
        out_shape=jax.ShapeDtypeStruct((NT, HID), jnp.float32),
    )(agg, w1, b1.reshape(1, -1), w2, b2.reshape(1, -1))


def _pe_embed(x, basis):
    p = x @ basis
    return jnp.concatenate([jnp.sin(p), jnp.cos(p)], axis=-1)


def _linear(x, w, b=None):
    y = x @ w
    return y + b if b is not None else y


def _layernorm(x, g, b, eps=1e-6):
    m = x.mean(-1, keepdims=True)
    v = ((x - m) ** 2).mean(-1, keepdims=True)
    return (x - m) / jnp.sqrt(v + eps) * g + b


def _attention(x, p):
    Bt, T, C = x.shape
    hd = C // HEADS
    qkv = _linear(x, p['qkv_w'], p['qkv_b']).reshape(
        Bt, T, 3, HEADS, hd).transpose(2, 0, 3, 1, 4)
    q, k, v = qkv[0], qkv[1], qkv[2]
    a = jax.nn.softmax((q @ k.transpose(0, 1, 3, 2)) * (hd ** -0.5), axis=-1)
    o = (a @ v).transpose(0, 2, 1, 3).reshape(Bt, T, C)
    return _linear(o, p['proj_w'], p['proj_b'])


def _block(x, p):
    h = x + p['g1'] * _attention(_layernorm(x, p['ln1_g'], p['ln1_b']), p)
    m = _linear(jax.nn.gelu(_linear(_layernorm(h, p['ln2_g'], p['ln2_b']),
                                    p['fc1_w'], p['fc1_b'])),
                p['fc2_w'], p['fc2_b'])
    return h + p['g2'] * m


def _transformer(x, pe, params):
    h = x + pe
    for p in params['blocks']:
        h = _block(h, p)
    return _layernorm(h, params['ln_f_g'], params['ln_f_b'])


def kernel(pc, idx, params, basis):
    pos = pc.reshape(B * N, 3)
    idx_flat = (idx + jnp.arange(B)[:, None] * N).reshape(-1)
    centers = pos[idx_flat]
    c = centers.reshape(B, A, 3)
    d2 = jnp.sum((c[:, :, None, :] - pc[:, None, :, :]) ** 2, axis=-1)
    _, nn_idx = lax.top_k(-d2, K)
    src = (nn_idx + jnp.arange(B)[:, None, None] * N).reshape(NT, K)
    nbr = pos[src]
    nbr = jnp.pad(nbr, ((0, 0), (0, KP - K), (0, 0)))

    agg = _edge_agg(nbr, centers, basis,
                    params['loc1_w'], params['loc1_b'],
                    params['loc2_w'], params['loc2_b'])
    x = _glob(agg, params['glob1_w'], params['glob1_b'],
              params['glob2_w'], params['glob2_b'])

    x = x.reshape(B, A, HID)
    center = centers.reshape(B, A, 3)

    def embed_in(v):
        return _linear(jnp.concatenate([v, _pe_embed(v, basis)], axis=-1),
                       params['embed_w'], params['embed_b'])

    ce = embed_in(center)
    x = _transformer(x, ce, params)
    dxyz = _linear(_layernorm(x, params['lnx_g'], params['lnx_b']),
                   params['xyz_w'])
    de = embed_in(dxyz)
    x = _transformer(x, ce + de, params)
    dtxyz = _linear(_layernorm(x, params['lnt_g'], params['lnt_b']),
                    params['txyz_w'])
    te = embed_in(dtxyz)
    x = _transformer(x, ce + de + te, params)
    sh2 = _linear(_layernorm(x, params['ln2d_g'], params['ln2d_b']),
                  params['sh2d_w'])
    s2e = _linear(sh2, params['sh2emb_w'], params['sh2emb_b'])
    x = _transformer(x, ce + de + te + s2e, params)
    sh3 = _linear(_layernorm(x, params['ln3d_g'], params['ln3d_b']),
                  params['sh3d_w'])
    return jnp.concatenate([center + dxyz, center + dtxyz, sh2, sh3],
                           axis=-1)


# speedup vs baseline: 1.1851x; 1.0057x over previous
import functools
from math import ceil

import jax
import jax.numpy as jnp
from jax import lax
from jax.experimental import pallas as pl

B, N, A = 8, 16384, 40
K = ceil(N / A)
KP = 416
HID = 192
HEADS = 6
DEPTH = 6
SH2 = 7
SH3 = 49
NT = B * A
CPB = 16
NEG = -1e30


def _bf(x):
    return x.astype(jnp.bfloat16).astype(jnp.float32)


def _pe_args(v, basis):
    return jnp.dot(_bf(v), _bf(basis), precision=lax.Precision.HIGHEST,
                   preferred_element_type=jnp.float32)


def _edge_kernel(nbr_ref, ctr_ref, basis_ref, w1_ref, b1_ref, w2_ref,
                 b2_ref, out_ref):
    basis = basis_ref[...]
    w1 = w1_ref[...]
    b1 = b1_ref[...]
    w2 = w2_ref[...]
    b2 = b2_ref[...]
    rowid = lax.broadcasted_iota(jnp.int32, (KP, 256), 0)
    valid = rowid < K
    rel = jnp.concatenate(
        [nbr_ref[0, i * KP:(i + 1) * KP, :] - ctr_ref[0, i, :]
         for i in range(CPB)], axis=0)
    p = _pe_args(rel, basis)
    feat = jnp.concatenate([rel, jnp.sin(p), jnp.cos(p)], axis=-1)
    l1 = jnp.dot(feat, w1, preferred_element_type=jnp.float32) + b1
    l1 = jnp.maximum(l1, 0.0)
    h = jnp.dot(l1, w2, preferred_element_type=jnp.float32) + b2
    for i in range(CPB):
        hc = jnp.where(valid, h[i * KP:(i + 1) * KP, :], NEG)
        out_ref[0, i, :] = jnp.max(hc, axis=0)


def _edge_agg(nbr, centers, basis, loc1_w, loc1_b, loc2_w, loc2_b):
    g = NT // CPB
    nbr_r = nbr.reshape(g, CPB * KP, 3)
    ctr_r = centers.reshape(g, CPB, 3)
    b1 = loc1_b.reshape(1, -1)
    b2 = loc2_b.reshape(1, -1)
    full = lambda a: pl.BlockSpec(a.shape, lambda i: (0,) * a.ndim)
    return pl.pallas_call(
        _edge_kernel,
        grid=(g,),
        in_specs=[
            pl.BlockSpec((1, CPB * KP, 3), lambda i: (i, 0, 0)),
            pl.BlockSpec((1, CPB, 3), lambda i: (i, 0, 0)),
            full(basis), full(loc1_w), full(b1), full(loc2_w), full(b2),
        ],
        out_specs=pl.BlockSpec((1, CPB, 256), lambda i: (i, 0, 0)),
        out_shape=jax.ShapeDtypeStruct((g, CPB, 256), jnp.float32),
    )(nbr_r, ctr_r, basis, loc1_w, b1, loc2_w, b2).reshape(NT, 256)


def _glob_kernel(agg_ref, w1_ref, b1_ref, w2_ref, b2_ref, out_ref):
    dot = functools.partial(jnp.dot, preferred_element_type=jnp.float32)
    x = jnp.maximum(dot(agg_ref[...], w1_ref[...]) + b1_ref[...], 0.0)
    out_ref[...] = dot(x, w2_ref[...]) + b2_ref[...]


def _glob(agg, w1, b1, w2, b2):
    return pl.pallas_call(
        _glob_kernel,
        out_shape=jax.ShapeDtypeStruct((NT, HID), jnp.float32),
    )(agg, w1, b1.reshape(1, -1), w2, b2.reshape(1, -1))


def _pe_embed(x, basis):
    p = x @ basis
    return jnp.concatenate([jnp.sin(p), jnp.cos(p)], axis=-1)


def _linear(x, w, b=None):
    y = x @ w
    return y + b if b is not None else y


def _layernorm(x, g, b, eps=1e-6):
    m = x.mean(-1, keepdims=True)
    v = ((x - m) ** 2).mean(-1, keepdims=True)
    return (x - m) / jnp.sqrt(v + eps) * g + b


def _attention(x, p):
    Bt, T, C = x.shape
    hd = C // HEADS
    qkv = _linear(x, p['qkv_w'], p['qkv_b']).reshape(
        Bt, T, 3, HEADS, hd).transpose(2, 0, 3, 1, 4)
    q, k, v = qkv[0], qkv[1], qkv[2]
    a = jax.nn.softmax((q @ k.transpose(0, 1, 3, 2)) * (hd ** -0.5), axis=-1)
    o = (a @ v).transpose(0, 2, 1, 3).reshape(Bt, T, C)
    return _linear(o, p['proj_w'], p['proj_b'])


def _block(x, p):
    h = x + p['g1'] * _attention(_layernorm(x, p['ln1_g'], p['ln1_b']), p)
    m = _linear(jax.nn.gelu(_linear(_layernorm(h, p['ln2_g'], p['ln2_b']),
                                    p['fc1_w'], p['fc1_b'])),
                p['fc2_w'], p['fc2_b'])
    return h + p['g2'] * m


def _transformer(x, pe, params):
    h = x + pe
    for p in params['blocks']:
        h = _block(h, p)
    return _layernorm(h, params['ln_f_g'], params['ln_f_b'])


def kernel(pc, idx, params, basis):
    pos = pc.reshape(B * N, 3)
    idx_flat = (idx + jnp.arange(B)[:, None] * N).reshape(-1)
    centers = pos[idx_flat]
    c = centers.reshape(B, A, 3)
    d2 = jnp.sum((c[:, :, None, :] - pc[:, None, :, :]) ** 2, axis=-1)
    _, nn_idx = lax.top_k(-d2, K)
    src = (nn_idx + jnp.arange(B)[:, None, None] * N).reshape(NT, K)
    nbr = pos[src]
    nbr = jnp.pad(nbr, ((0, 0), (0, KP - K), (0, 0)))

    agg = _edge_agg(nbr, centers, basis,
                    params['loc1_w'], params['loc1_b'],
                    params['loc2_w'], params['loc2_b'])
    x = _glob(agg, params['glob1_w'], params['glob1_b'],
              params['glob2_w'], params['glob2_b'])

    x = x.reshape(B, A, HID)
    center = centers.reshape(B, A, 3)

    def embed_in(v):
        return _linear(jnp.concatenate([v, _pe_embed(v, basis)], axis=-1),
                       params['embed_w'], params['embed_b'])

    ce = embed_in(center)
    x = _transformer(x, ce, params)
    dxyz = _linear(_layernorm(x, params['lnx_g'], params['lnx_b']),
                   params['xyz_w'])
    de = embed_in(dxyz)
    x = _transformer(x, ce + de, params)
    dtxyz = _linear(_layernorm(x, params['lnt_g'], params['lnt_b']),
                    params['txyz_w'])
    te = embed_in(dtxyz)
    x = _transformer(x, ce + de + te, params)
    sh2 = _linear(_layernorm(x, params['ln2d_g'], params['ln2d_b']),
                  params['sh2d_w'])
    s2e = _linear(sh2, params['sh2emb_w'], params['sh2emb_b'])
    x = _transformer(x, ce + de + te + s2e, params)
    sh3 = _linear(_layernorm(x, params['ln3d_g'], params['ln3d_b']),
                  params['sh3d_w'])
    return jnp.concatenate([center + dxyz, center + dtxyz, sh2, sh3],
                           axis=-1)
